# baseline (speedup 1.0000x reference)
import jax
import jax.numpy as jnp
from jax.experimental import pallas as pl
from jax.experimental.pallas import tpu as pltpu

B, N, H, K = 8, 1024, 64, 9
BN = 512


def _body(idx_ref, m_ref, k_ref, v_ref, o_ref):
    b = pl.program_id(0)
    j = pl.program_id(1)
    o_ref[...] = m_ref[...]
    lane_iota = jax.lax.broadcasted_iota(jnp.int32, (BN,), 0)
    for kk in range(K):
        idx = idx_ref[b, kk]
        inb = (idx >= j * BN) & (idx < (j + 1) * BN)

        @pl.when(inb)
        def _(idx=idx):
            lane = idx - j * BN
            onehot = (lane_iota == lane).astype(jnp.float32)
            kcol = jnp.sum(k_ref[0] * onehot[None, :], axis=-1)
            vcol = jnp.sum(v_ref[0] * onehot[None, :], axis=-1)
            outer = kcol[:, None] * vcol[None, :]
            o_ref[0] += outer[:, :, None] * onehot[None, None, :]


@jax.jit
def kernel(M, M_k, M_v, indices_update):
    idx = indices_update.astype(jnp.int32)
    Mt = jnp.transpose(M, (0, 2, 3, 1))
    Kt = jnp.transpose(M_k, (0, 2, 1))
    Vt = jnp.transpose(M_v, (0, 2, 1))
    out_t = pl.pallas_call(
        _body,
        grid=(B, N // BN),
        in_specs=[
            pl.BlockSpec(memory_space=pltpu.SMEM),
            pl.BlockSpec((1, H, H, BN), lambda b, j: (b, 0, 0, j)),
            pl.BlockSpec((1, H, BN), lambda b, j: (b, 0, j)),
            pl.BlockSpec((1, H, BN), lambda b, j: (b, 0, j)),
        ],
        out_specs=pl.BlockSpec((1, H, H, BN), lambda b, j: (b, 0, 0, j)),
        out_shape=jax.ShapeDtypeStruct((B, H, H, N), jnp.float32),
        compiler_params=pltpu.CompilerParams(
            dimension_semantics=("parallel", "parallel"),
        ),
    )(idx, Mt, Kt, Vt)
    return jnp.transpose(out_t, (0, 3, 1, 2))


# speedup vs baseline: 1.0116x; 1.0116x over previous
import jax
import jax.numpy as jnp
from jax.experimental import pallas as pl
from jax.experimental.pallas import tpu as pltpu

B, N, H, K = 8, 1024, 64, 9
BN = 256
TW = 128


def _body(idx_ref, m_ref, k_ref, v_ref, o_ref):
    b = pl.program_id(0)
    j = pl.program_id(1)
    o_ref[...] = m_ref[...]
    lane_iota = jax.lax.broadcasted_iota(jnp.int32, (BN,), 0)
    win_iota = jax.lax.broadcasted_iota(jnp.int32, (TW,), 0)
    for kk in range(K):
        idx = idx_ref[b, kk]
        inb = (idx >= j * BN) & (idx < (j + 1) * BN)

        @pl.when(inb)
        def _(idx=idx):
            lane = idx - j * BN
            onehot = (lane_iota == lane).astype(jnp.float32)
            kcol = jnp.sum(k_ref[0] * onehot[None, :], axis=-1)
            vcol = jnp.sum(v_ref[0] * onehot[None, :], axis=-1)
            outer = kcol[:, None] * vcol[None, :]
            t = lane // TW
            wlane = lane - t * TW
            onehot_w = (win_iota == wlane).astype(jnp.float32)
            o_ref[0, :, :, pl.ds(t * TW, TW)] += (
                outer[:, :, None] * onehot_w[None, None, :]
            )


@jax.jit
def kernel(M, M_k, M_v, indices_update):
    idx = indices_update.astype(jnp.int32)
    Mt = jnp.transpose(M, (0, 2, 3, 1))
    Kt = jnp.transpose(M_k, (0, 2, 1))
    Vt = jnp.transpose(M_v, (0, 2, 1))
    out_t = pl.pallas_call(
        _body,
        grid=(B, N // BN),
        in_specs=[
            pl.BlockSpec(memory_space=pltpu.SMEM),
            pl.BlockSpec((1, H, H, BN), lambda b, j: (b, 0, 0, j)),
            pl.BlockSpec((1, H, BN), lambda b, j: (b, 0, j)),
            pl.BlockSpec((1, H, BN), lambda b, j: (b, 0, j)),
        ],
        out_specs=pl.BlockSpec((1, H, H, BN), lambda b, j: (b, 0, 0, j)),
        out_shape=jax.ShapeDtypeStruct((B, H, H, N), jnp.float32),
        compiler_params=pltpu.CompilerParams(
            dimension_semantics=("parallel", "parallel"),
        ),
    )(idx, Mt, Kt, Vt)
    return jnp.transpose(out_t, (0, 3, 1, 2))
